# baseline (speedup 1.0000x reference)
import functools

import jax
import jax.numpy as jnp
from jax import lax
from jax.experimental import pallas as pl
from jax.experimental.pallas import tpu as pltpu
from jax.experimental.pallas import tpu_sc as plsc

_MAX_RANGE = 1000
_BATCH = 16384
_NUM_CORES = 2
_NUM_SUBCORES = 16
_NW = _NUM_CORES * _NUM_SUBCORES
_CHUNK = _BATCH // _NW
_NSEG = _CHUNK // 128
_MAGIC = 8388608.0


@functools.partial(
    pl.kernel,
    mesh=plsc.VectorSubcoreMesh(core_axis_name="c", subcore_axis_name="s"),
    out_type=jax.ShapeDtypeStruct((_BATCH,), jnp.float32),
    scratch_types=[
        pltpu.VMEM((_CHUNK,), jnp.float32),
        pltpu.VMEM((_NSEG, 128), jnp.int32),
        pltpu.VMEM((_CHUNK,), jnp.float32),
        pltpu.SemaphoreType.DMA,
    ],
)
def _gather(x_hbm, param_hbm, out_hbm, x_v, idx_v, gat_v, sem):
    wid = lax.axis_index("s") * _NUM_CORES + lax.axis_index("c")
    base = wid * _CHUNK
    pltpu.sync_copy(x_hbm.at[pl.ds(base, _CHUNK)], x_v)
    lane = lax.iota(jnp.int32, 16)
    for j in range(_NSEG):
        for c in range(128 // 16):
            off = j * 128 + c * 16
            xv = x_v[pl.ds(off, 16)]
            xs = xv * float(_MAX_RANGE - 1)
            rounded = (xs + _MAGIC) - _MAGIC
            col = rounded.astype(jnp.int32)
            idx_v[j, pl.ds(c * 16, 16)] = (base + off + lane) * _MAX_RANGE + col
    copies = [
        pltpu.async_copy(param_hbm.at[idx_v.at[j]],
                         gat_v.at[pl.ds(j * 128, 128)], sem)
        for j in range(_NSEG)
    ]
    for cp in copies:
        cp.wait()
    pltpu.sync_copy(gat_v, out_hbm.at[pl.ds(base, _CHUNK)])


def kernel(x, param):
    pflat = param.reshape(_BATCH * _MAX_RANGE)
    out = _gather(x, pflat)
    return out.reshape(_BATCH, 1, 1)
